# baseline (speedup 1.0000x reference)
import functools

import jax
import jax.numpy as jnp
from jax.experimental import pallas as pl
from jax.experimental.pallas import tpu as pltpu
from jax.experimental.pallas import tpu_sc as plsc

_N_EMB = 1024
_DIM = 64
_B = 32
_RPB = 1024
_N_ROWS = _B * _RPB
_COMMIT = 0.25


def _vq_body(lat_ref, w_ref, dist_ref, enc_ref, idx_ref, cnt_ref, loss_ref,
             perp_ref):
    i = pl.program_id(0)
    lm = lat_ref[0]
    w = w_ref[...]
    mm = jax.lax.dot_general(lm, w, (((0,), (1,)), ((), ())),
                             preferred_element_type=jnp.float32)
    rowsq = jnp.sum(lm * lm, axis=0).reshape(_RPB, 1)
    wsq = jnp.sum(w * w, axis=1).reshape(1, _N_EMB)
    dist = (rowsq + wsq) - 2.0 * mm
    dist_ref[...] = dist

    minv = jnp.min(dist, axis=1, keepdims=True)
    cols = jax.lax.broadcasted_iota(jnp.int32, (_RPB, _N_EMB), 1)
    idx = jnp.min(jnp.where(dist == minv, cols, _N_EMB), axis=1, keepdims=True)
    enc = (cols == idx).astype(jnp.float32)
    enc_ref[...] = enc
    iota_i = jax.lax.broadcasted_iota(jnp.int32, (1, _N_EMB), 1)
    dims = (((1,), (1,)), ((), ()))
    hi_d = jax.lax.dot_general((iota_i >> 8).astype(jnp.float32), enc, dims,
                               preferred_element_type=jnp.float32)
    lo_d = jax.lax.dot_general((iota_i & 255).astype(jnp.float32), enc, dims,
                               preferred_element_type=jnp.float32)
    idxf = hi_d * 256.0 + lo_d
    idx_ref[...] = idxf.astype(jnp.int32).reshape(8, 128)

    pcnt = jnp.sum(enc, axis=0).reshape(1, _N_EMB)
    ploss = jnp.sum(minv)

    @pl.when(i == 0)
    def _init():
        cnt_ref[...] = pcnt
        loss_ref[0, 0] = ploss

    @pl.when(i > 0)
    def _acc():
        cnt_ref[...] += pcnt
        loss_ref[0, 0] += ploss

    @pl.when(i == _B - 1)
    def _fin():
        avg = cnt_ref[...] * (1.0 / _N_ROWS)
        perp_ref[0, 0] = jnp.exp(-jnp.sum(avg * jnp.log(avg + 1e-10)))
        loss_ref[0, 0] = loss_ref[0, 0] * (_COMMIT / (_N_ROWS * _DIM))


_IN_SPECS = [
    pl.BlockSpec((1, _DIM, _RPB), lambda i: (i, 0, 0)),
    pl.BlockSpec((_N_EMB, _DIM), lambda i: (0, 0)),
]
_OUT_SPECS = [
    pl.BlockSpec((_RPB, _N_EMB), lambda i: (i, 0)),
    pl.BlockSpec((_RPB, _N_EMB), lambda i: (i, 0)),
    pl.BlockSpec((8, 128), lambda i: (i, 0)),
    pl.BlockSpec((1, _N_EMB), lambda i: (0, 0)),
    pl.BlockSpec((1, 1), lambda i: (0, 0), memory_space=pltpu.SMEM),
    pl.BlockSpec((1, 1), lambda i: (0, 0), memory_space=pltpu.SMEM),
]
_OUT_SHAPE = [
    jax.ShapeDtypeStruct((_N_ROWS, _N_EMB), jnp.float32),
    jax.ShapeDtypeStruct((_N_ROWS, _N_EMB), jnp.float32),
    jax.ShapeDtypeStruct((_N_ROWS // 128, 128), jnp.int32),
    jax.ShapeDtypeStruct((1, _N_EMB), jnp.float32),
    jax.ShapeDtypeStruct((1, 1), jnp.float32),
    jax.ShapeDtypeStruct((1, 1), jnp.float32),
]


_NW = 32
_BPW = _N_ROWS // _NW


_RC = 512


@functools.lru_cache(maxsize=None)
def _sc_gather_fn():
    @functools.partial(
        pl.kernel,
        mesh=plsc.VectorSubcoreMesh(core_axis_name="c", subcore_axis_name="s"),
        out_type=jax.ShapeDtypeStruct((_N_ROWS, 128), jnp.float32),
        scratch_types=[
            pltpu.VMEM((_BPW,), jnp.int32),
            pltpu.VMEM((_RC, 128), jnp.float32),
            pltpu.SemaphoreType.DMA,
        ],
    )
    def _sc_gather(wpad_hbm, idx_hbm, out_hbm, idx_v, rows_v, sem):
        wid = jax.lax.axis_index("s") * 2 + jax.lax.axis_index("c")
        base = wid * _BPW
        pltpu.sync_copy(idx_hbm.at[pl.ds(base, _BPW)], idx_v)
        for r0 in range(0, _BPW, _RC):
            pltpu.async_copy(
                wpad_hbm.at[idx_v.at[pl.ds(r0, _RC)]], rows_v, sem).wait()
            pltpu.sync_copy(rows_v, out_hbm.at[pl.ds(base + r0, _RC), :])

    return _sc_gather


def kernel(latent, embedding_weight):
    lat3 = latent.reshape(_B, _DIM, _RPB)
    dist, enc, idx, _cnt, loss, perp = pl.pallas_call(
        _vq_body,
        grid=(_B,),
        in_specs=_IN_SPECS,
        out_specs=_OUT_SPECS,
        out_shape=_OUT_SHAPE,
        compiler_params=pltpu.CompilerParams(
            dimension_semantics=("arbitrary",)),
    )(lat3, embedding_weight)
    wpad = jnp.concatenate(
        [embedding_weight,
         jnp.zeros((_N_EMB, 128 - _DIM), embedding_weight.dtype)], axis=1)
    q128 = _sc_gather_fn()(wpad, idx.reshape(_N_ROWS))
    quantized_out = jnp.transpose(
        q128.reshape(_B, 32, 32, 128)[:, :, :, :_DIM], (0, 3, 1, 2))
    return quantized_out, loss[0, 0], perp[0, 0], enc, dist


# speedup vs baseline: 1.0523x; 1.0523x over previous
import functools

import jax
import jax.numpy as jnp
from jax.experimental import pallas as pl
from jax.experimental.pallas import tpu as pltpu
from jax.experimental.pallas import tpu_sc as plsc

_N_EMB = 1024
_DIM = 64
_B = 32
_RPB = 1024
_N_ROWS = _B * _RPB
_COMMIT = 0.25


def _vq_body(lat_ref, w_ref, dist_ref, enc_ref, idx_ref, cnt_ref, loss_ref,
             perp_ref):
    i = pl.program_id(0)
    lm = lat_ref[0]
    w = w_ref[...]
    mm = jax.lax.dot_general(lm, w, (((0,), (1,)), ((), ())),
                             preferred_element_type=jnp.float32)
    rowsq = jnp.sum(lm * lm, axis=0).reshape(_RPB, 1)
    wsq = jnp.sum(w * w, axis=1).reshape(1, _N_EMB)
    dist = (rowsq + wsq) - 2.0 * mm
    dist_ref[...] = dist

    minv = jnp.min(dist, axis=1, keepdims=True)
    cols = jax.lax.broadcasted_iota(jnp.int32, (_RPB, _N_EMB), 1)
    idx = jnp.min(jnp.where(dist == minv, cols, _N_EMB), axis=1, keepdims=True)
    enc = (cols == idx).astype(jnp.float32)
    enc_ref[...] = enc
    idx_ref[...] = idx

    pcnt = jnp.sum(enc, axis=0).reshape(1, _N_EMB)
    ploss = jnp.sum(minv)

    @pl.when(i == 0)
    def _init():
        cnt_ref[...] = pcnt
        loss_ref[0, 0] = ploss

    @pl.when(i > 0)
    def _acc():
        cnt_ref[...] += pcnt
        loss_ref[0, 0] += ploss

    @pl.when(i == _B - 1)
    def _fin():
        avg = cnt_ref[...] * (1.0 / _N_ROWS)
        perp_ref[0, 0] = jnp.exp(-jnp.sum(avg * jnp.log(avg + 1e-10)))
        loss_ref[0, 0] = loss_ref[0, 0] * (_COMMIT / (_N_ROWS * _DIM))


_IN_SPECS = [
    pl.BlockSpec((1, _DIM, _RPB), lambda i: (i, 0, 0)),
    pl.BlockSpec((_N_EMB, _DIM), lambda i: (0, 0)),
]
_OUT_SPECS = [
    pl.BlockSpec((_RPB, _N_EMB), lambda i: (i, 0)),
    pl.BlockSpec((_RPB, _N_EMB), lambda i: (i, 0)),
    pl.BlockSpec((_RPB, 1), lambda i: (i, 0)),
    pl.BlockSpec((1, _N_EMB), lambda i: (0, 0)),
    pl.BlockSpec((1, 1), lambda i: (0, 0), memory_space=pltpu.SMEM),
    pl.BlockSpec((1, 1), lambda i: (0, 0), memory_space=pltpu.SMEM),
]
_OUT_SHAPE = [
    jax.ShapeDtypeStruct((_N_ROWS, _N_EMB), jnp.float32),
    jax.ShapeDtypeStruct((_N_ROWS, _N_EMB), jnp.float32),
    jax.ShapeDtypeStruct((_N_ROWS, 1), jnp.int32),
    jax.ShapeDtypeStruct((1, _N_EMB), jnp.float32),
    jax.ShapeDtypeStruct((1, 1), jnp.float32),
    jax.ShapeDtypeStruct((1, 1), jnp.float32),
]


_NW = 32
_BPW = _N_ROWS // _NW


_RC = 512


@functools.lru_cache(maxsize=None)
def _sc_gather_fn():
    @functools.partial(
        pl.kernel,
        mesh=plsc.VectorSubcoreMesh(core_axis_name="c", subcore_axis_name="s"),
        out_type=jax.ShapeDtypeStruct((_N_ROWS, 128), jnp.float32),
        scratch_types=[
            pltpu.VMEM((_BPW,), jnp.int32),
            pltpu.VMEM((_RC, 128), jnp.float32),
            pltpu.SemaphoreType.DMA,
        ],
    )
    def _sc_gather(wpad_hbm, idx_hbm, out_hbm, idx_v, rows_v, sem):
        wid = jax.lax.axis_index("s") * 2 + jax.lax.axis_index("c")
        base = wid * _BPW
        pltpu.sync_copy(idx_hbm.at[pl.ds(base, _BPW)], idx_v)
        for r0 in range(0, _BPW, _RC):
            pltpu.async_copy(
                wpad_hbm.at[idx_v.at[pl.ds(r0, _RC)]], rows_v, sem).wait()
            pltpu.sync_copy(rows_v, out_hbm.at[pl.ds(base + r0, _RC), :])

    return _sc_gather


def kernel(latent, embedding_weight):
    lat3 = latent.reshape(_B, _DIM, _RPB)
    dist, enc, idx, _cnt, loss, perp = pl.pallas_call(
        _vq_body,
        grid=(_B,),
        in_specs=_IN_SPECS,
        out_specs=_OUT_SPECS,
        out_shape=_OUT_SHAPE,
        compiler_params=pltpu.CompilerParams(
            dimension_semantics=("arbitrary",)),
    )(lat3, embedding_weight)
    wpad = jnp.concatenate(
        [embedding_weight,
         jnp.zeros((_N_EMB, 128 - _DIM), embedding_weight.dtype)], axis=1)
    q128 = _sc_gather_fn()(wpad, idx.reshape(_N_ROWS))
    quantized_out = jnp.transpose(
        q128.reshape(_B, 32, 32, 128)[:, :, :, :_DIM], (0, 3, 1, 2))
    return quantized_out, loss[0, 0], perp[0, 0], enc, dist


# speedup vs baseline: 1.0877x; 1.0337x over previous
import functools

import jax
import jax.numpy as jnp
from jax.experimental import pallas as pl
from jax.experimental.pallas import tpu as pltpu
from jax.experimental.pallas import tpu_sc as plsc

_N_EMB = 1024
_DIM = 64
_B = 32
_RPB = 1024
_N_ROWS = _B * _RPB
_COMMIT = 0.25


def _vq_body(lat_ref, w_ref, dist_ref, enc_ref, idx_ref, cnt_ref, loss_ref,
             perp_ref):
    i = pl.program_id(0)
    fl = lat_ref[...]
    w = w_ref[...]
    mm = jax.lax.dot_general(fl, w, (((1,), (1,)), ((), ())),
                             preferred_element_type=jnp.float32)
    rowsq = jnp.sum(fl * fl, axis=1, keepdims=True)
    wsq = jnp.sum(w * w, axis=1).reshape(1, _N_EMB)
    dist = (rowsq + wsq) - 2.0 * mm
    dist_ref[...] = dist

    minv = jnp.min(dist, axis=1, keepdims=True)
    cols = jax.lax.broadcasted_iota(jnp.int32, (_RPB, _N_EMB), 1)
    idx = jnp.min(jnp.where(dist == minv, cols, _N_EMB), axis=1, keepdims=True)
    enc = (cols == idx).astype(jnp.float32)
    enc_ref[...] = enc
    idx_ref[...] = idx

    pcnt = jnp.sum(enc, axis=0).reshape(1, _N_EMB)
    ploss = jnp.sum(minv)

    @pl.when(i == 0)
    def _init():
        cnt_ref[...] = pcnt
        loss_ref[0, 0] = ploss

    @pl.when(i > 0)
    def _acc():
        cnt_ref[...] += pcnt
        loss_ref[0, 0] += ploss

    @pl.when(i == _B - 1)
    def _fin():
        avg = cnt_ref[...] * (1.0 / _N_ROWS)
        perp_ref[0, 0] = jnp.exp(-jnp.sum(avg * jnp.log(avg + 1e-10)))
        loss_ref[0, 0] = loss_ref[0, 0] * (_COMMIT / (_N_ROWS * _DIM))


_IN_SPECS = [
    pl.BlockSpec((_RPB, _DIM), lambda i: (i, 0)),
    pl.BlockSpec((_N_EMB, _DIM), lambda i: (0, 0)),
]
_OUT_SPECS = [
    pl.BlockSpec((_RPB, _N_EMB), lambda i: (i, 0)),
    pl.BlockSpec((_RPB, _N_EMB), lambda i: (i, 0)),
    pl.BlockSpec((_RPB, 1), lambda i: (i, 0)),
    pl.BlockSpec((1, _N_EMB), lambda i: (0, 0)),
    pl.BlockSpec((1, 1), lambda i: (0, 0), memory_space=pltpu.SMEM),
    pl.BlockSpec((1, 1), lambda i: (0, 0), memory_space=pltpu.SMEM),
]
_OUT_SHAPE = [
    jax.ShapeDtypeStruct((_N_ROWS, _N_EMB), jnp.float32),
    jax.ShapeDtypeStruct((_N_ROWS, _N_EMB), jnp.float32),
    jax.ShapeDtypeStruct((_N_ROWS, 1), jnp.int32),
    jax.ShapeDtypeStruct((1, _N_EMB), jnp.float32),
    jax.ShapeDtypeStruct((1, 1), jnp.float32),
    jax.ShapeDtypeStruct((1, 1), jnp.float32),
]


_NW = 32
_BPW = _N_ROWS // _NW


_RC = 512


@functools.lru_cache(maxsize=None)
def _sc_gather_fn():
    @functools.partial(
        pl.kernel,
        mesh=plsc.VectorSubcoreMesh(core_axis_name="c", subcore_axis_name="s"),
        out_type=jax.ShapeDtypeStruct((_N_ROWS, 128), jnp.float32),
        scratch_types=[
            pltpu.VMEM((_BPW,), jnp.int32),
            pltpu.VMEM((_RC, 128), jnp.float32),
            pltpu.SemaphoreType.DMA,
        ],
    )
    def _sc_gather(wpad_hbm, idx_hbm, out_hbm, idx_v, rows_v, sem):
        wid = jax.lax.axis_index("s") * 2 + jax.lax.axis_index("c")
        base = wid * _BPW
        pltpu.sync_copy(idx_hbm.at[pl.ds(base, _BPW)], idx_v)
        for r0 in range(0, _BPW, _RC):
            pltpu.async_copy(
                wpad_hbm.at[idx_v.at[pl.ds(r0, _RC)]], rows_v, sem).wait()
            pltpu.sync_copy(rows_v, out_hbm.at[pl.ds(base + r0, _RC), :])

    return _sc_gather


def kernel(latent, embedding_weight):
    flat = jnp.transpose(latent, (0, 2, 3, 1)).reshape(_N_ROWS, _DIM)
    dist, enc, idx, _cnt, loss, perp = pl.pallas_call(
        _vq_body,
        grid=(_B,),
        in_specs=_IN_SPECS,
        out_specs=_OUT_SPECS,
        out_shape=_OUT_SHAPE,
        compiler_params=pltpu.CompilerParams(
            dimension_semantics=("arbitrary",)),
    )(flat, embedding_weight)
    wpad = jnp.concatenate(
        [embedding_weight,
         jnp.zeros((_N_EMB, 128 - _DIM), embedding_weight.dtype)], axis=1)
    q128 = _sc_gather_fn()(wpad, idx.reshape(_N_ROWS))
    quantized_out = jnp.transpose(
        q128.reshape(_B, 32, 32, 128)[:, :, :, :_DIM], (0, 3, 1, 2))
    return quantized_out, loss[0, 0], perp[0, 0], enc, dist


# speedup vs baseline: 1.2030x; 1.1060x over previous
import functools

import jax
import jax.numpy as jnp
from jax.experimental import pallas as pl
from jax.experimental.pallas import tpu as pltpu
from jax.experimental.pallas import tpu_sc as plsc

_N_EMB = 1024
_DIM = 64
_B = 32
_RPB = 1024
_N_ROWS = _B * _RPB
_COMMIT = 0.25


def _vq_body(lat_ref, w_ref, dist_ref, enc_ref, idx_ref, cnt_ref, loss_ref,
             perp_ref):
    i = pl.program_id(0)
    fl = lat_ref[...]
    w = w_ref[...]
    mm = jax.lax.dot_general(fl, w, (((1,), (1,)), ((), ())),
                             preferred_element_type=jnp.float32)
    rowsq = jnp.sum(fl * fl, axis=1, keepdims=True)
    wsq = jnp.sum(w * w, axis=1).reshape(1, _N_EMB)
    dist = (rowsq + wsq) - 2.0 * mm
    dist_ref[...] = dist

    minv = jnp.min(dist, axis=1, keepdims=True)
    cols = jax.lax.broadcasted_iota(jnp.int32, (_RPB, _N_EMB), 1)
    idx = jnp.min(jnp.where(dist == minv, cols, _N_EMB), axis=1, keepdims=True)
    enc = (cols == idx).astype(jnp.float32)
    enc_ref[...] = enc
    idx_ref[0] = jnp.transpose(idx, (1, 0))

    pcnt = jnp.sum(enc, axis=0).reshape(1, _N_EMB)
    ploss = jnp.sum(minv)

    @pl.when(i == 0)
    def _init():
        cnt_ref[...] = pcnt
        loss_ref[0, 0] = ploss

    @pl.when(i > 0)
    def _acc():
        cnt_ref[...] += pcnt
        loss_ref[0, 0] += ploss

    @pl.when(i == _B - 1)
    def _fin():
        avg = cnt_ref[...] * (1.0 / _N_ROWS)
        perp_ref[0, 0] = jnp.exp(-jnp.sum(avg * jnp.log(avg + 1e-10)))
        loss_ref[0, 0] = loss_ref[0, 0] * (_COMMIT / (_N_ROWS * _DIM))


_IN_SPECS = [
    pl.BlockSpec((_RPB, _DIM), lambda i: (i, 0)),
    pl.BlockSpec((_N_EMB, _DIM), lambda i: (0, 0)),
]
_OUT_SPECS = [
    pl.BlockSpec((_RPB, _N_EMB), lambda i: (i, 0)),
    pl.BlockSpec((_RPB, _N_EMB), lambda i: (i, 0)),
    pl.BlockSpec((1, 1, _RPB), lambda i: (i, 0, 0)),
    pl.BlockSpec((1, _N_EMB), lambda i: (0, 0)),
    pl.BlockSpec((1, 1), lambda i: (0, 0), memory_space=pltpu.SMEM),
    pl.BlockSpec((1, 1), lambda i: (0, 0), memory_space=pltpu.SMEM),
]
_OUT_SHAPE = [
    jax.ShapeDtypeStruct((_N_ROWS, _N_EMB), jnp.float32),
    jax.ShapeDtypeStruct((_N_ROWS, _N_EMB), jnp.float32),
    jax.ShapeDtypeStruct((_B, 1, _RPB), jnp.int32),
    jax.ShapeDtypeStruct((1, _N_EMB), jnp.float32),
    jax.ShapeDtypeStruct((1, 1), jnp.float32),
    jax.ShapeDtypeStruct((1, 1), jnp.float32),
]


_NW = 32
_BPW = _N_ROWS // _NW


_RC = 512


@functools.lru_cache(maxsize=None)
def _sc_gather_fn():
    @functools.partial(
        pl.kernel,
        mesh=plsc.VectorSubcoreMesh(core_axis_name="c", subcore_axis_name="s"),
        out_type=jax.ShapeDtypeStruct((_N_ROWS, 128), jnp.float32),
        scratch_types=[
            pltpu.VMEM((_BPW,), jnp.int32),
            pltpu.VMEM((_RC, 128), jnp.float32),
            pltpu.SemaphoreType.DMA,
        ],
    )
    def _sc_gather(wpad_hbm, idx_hbm, out_hbm, idx_v, rows_v, sem):
        wid = jax.lax.axis_index("s") * 2 + jax.lax.axis_index("c")
        base = wid * _BPW
        pltpu.sync_copy(idx_hbm.at[pl.ds(base, _BPW)], idx_v)
        for r0 in range(0, _BPW, _RC):
            pltpu.async_copy(
                wpad_hbm.at[idx_v.at[pl.ds(r0, _RC)]], rows_v, sem).wait()
            pltpu.sync_copy(rows_v, out_hbm.at[pl.ds(base + r0, _RC), :])

    return _sc_gather


def kernel(latent, embedding_weight):
    flat = jnp.transpose(latent, (0, 2, 3, 1)).reshape(_N_ROWS, _DIM)
    dist, enc, idx, _cnt, loss, perp = pl.pallas_call(
        _vq_body,
        grid=(_B,),
        in_specs=_IN_SPECS,
        out_specs=_OUT_SPECS,
        out_shape=_OUT_SHAPE,
        compiler_params=pltpu.CompilerParams(
            dimension_semantics=("arbitrary",)),
    )(flat, embedding_weight)
    wpad = jnp.concatenate(
        [embedding_weight,
         jnp.zeros((_N_EMB, 128 - _DIM), embedding_weight.dtype)], axis=1)
    q128 = _sc_gather_fn()(wpad, idx.reshape(_N_ROWS))
    quantized_out = jnp.transpose(
        q128.reshape(_B, 32, 32, 128)[:, :, :, :_DIM], (0, 3, 1, 2))
    return quantized_out, loss[0, 0], perp[0, 0], enc, dist
